# baseline (speedup 1.0000x reference)
import functools

import jax
import jax.numpy as jnp
from jax import lax
from jax.experimental import pallas as pl
from jax.experimental.pallas import tpu as pltpu
from jax.experimental.pallas import tpu_sc as plsc

B = 4096
L = 200
DIM = 64
N = B * L
ROW = 128
NROWS = N // ROW
NC = 2
NS = 16
NW = NC * NS
ROWS_PER_W = NROWS // NW
NBUF = 2
CH = 4
NCHUNK = ROWS_PER_W // CH
NITER = NCHUNK // NBUF


@jax.jit
def _embed(table, idx):
    mesh = plsc.VectorSubcoreMesh(core_axis_name="c", subcore_axis_name="s")

    @functools.partial(
        pl.kernel,
        mesh=mesh,
        out_type=jax.ShapeDtypeStruct((N, DIM), jnp.float32),
        scratch_types=[
            pltpu.VMEM((NBUF, CH, ROW), jnp.int32),
            pltpu.VMEM((NBUF, CH * ROW, DIM), jnp.float32),
            pltpu.SemaphoreType.DMA,
            pltpu.SemaphoreType.DMA,
            pltpu.SemaphoreType.DMA,
            pltpu.SemaphoreType.DMA,
        ],
        compiler_params=pltpu.CompilerParams(use_tc_tiling_on_sc=False),
    )
    def emb(table_hbm, idx_hbm, out_hbm, idx_v, rows_v, g0, g1, s0, s1):
        gsem = (g0, g1)
        ssem = (s0, s1)
        wid = lax.axis_index("s") * NC + lax.axis_index("c")
        row_base = wid * ROWS_PER_W

        def fire_gathers(c, b):
            r0 = row_base + c * CH
            pltpu.sync_copy(idx_hbm.at[pl.ds(r0, CH)], idx_v.at[b])
            for j in range(CH):
                pltpu.async_copy(
                    table_hbm.at[idx_v.at[b].at[j]],
                    rows_v.at[b].at[pl.ds(j * ROW, ROW)],
                    gsem[b],
                )

        def wait_gathers(b):
            for j in range(CH):
                pltpu.make_async_copy(
                    table_hbm.at[idx_v.at[b].at[j]],
                    rows_v.at[b].at[pl.ds(j * ROW, ROW)],
                    gsem[b],
                ).wait()

        for b in range(NBUF):
            fire_gathers(b, b)

        def body(g, carry):
            for b in range(NBUF):
                c = g * NBUF + b
                f0 = (row_base + c * CH) * ROW
                wait_gathers(b)
                st = pltpu.async_copy(
                    rows_v.at[b], out_hbm.at[pl.ds(f0, CH * ROW)], ssem[b]
                )
                st.wait()
                nxt = c + NBUF

                @pl.when(nxt < NCHUNK)
                def _():
                    fire_gathers(nxt, b)

            return carry

        lax.fori_loop(0, NITER, body, 0)

    return emb(table, idx)


def kernel(x, table):
    idx = x.reshape(NROWS, ROW)
    out = _embed(table, idx)
    return out.reshape(B, L, DIM)


# speedup vs baseline: 1.2270x; 1.2270x over previous
import functools

import jax
import jax.numpy as jnp
from jax import lax
from jax.experimental import pallas as pl
from jax.experimental.pallas import tpu as pltpu
from jax.experimental.pallas import tpu_sc as plsc

B = 4096
L = 200
DIM = 64
PDIM = 128
N = B * L
ROW = 256
NROWS = N // ROW
NC = 2
NS = 16
NW = NC * NS
ROWS_PER_W = NROWS // NW
NBUF = 2
CH = 1
NCHUNK = ROWS_PER_W // CH
NITER = NCHUNK // NBUF


@jax.jit
def _embed(table_p, idx):
    mesh = plsc.VectorSubcoreMesh(core_axis_name="c", subcore_axis_name="s")

    @functools.partial(
        pl.kernel,
        mesh=mesh,
        out_type=jax.ShapeDtypeStruct((N, PDIM), jnp.float32),
        scratch_types=[
            pltpu.VMEM((NBUF, CH, ROW), jnp.int32),
            pltpu.VMEM((NBUF, CH * ROW, PDIM), jnp.float32),
            pltpu.SemaphoreType.DMA,
            pltpu.SemaphoreType.DMA,
            pltpu.SemaphoreType.DMA,
            pltpu.SemaphoreType.DMA,
        ],
        compiler_params=pltpu.CompilerParams(use_tc_tiling_on_sc=True),
    )
    def emb(table_hbm, idx_hbm, out_hbm, idx_v, rows_v,
            g0, g1, s0, s1):
        gsem = (g0, g1)
        ssem = (s0, s1)
        wid = lax.axis_index("s") * NC + lax.axis_index("c")
        row_base = wid * ROWS_PER_W

        def fire_gathers(c, b):
            r0 = row_base + c * CH
            pltpu.sync_copy(idx_hbm.at[pl.ds(r0, CH)], idx_v.at[b])
            for j in range(CH):
                pltpu.async_copy(
                    table_hbm.at[idx_v.at[b].at[j]],
                    rows_v.at[b].at[pl.ds(j * ROW, ROW)],
                    gsem[b],
                )

        def wait_gathers(b):
            for j in range(CH):
                pltpu.make_async_copy(
                    table_hbm.at[idx_v.at[b].at[j]],
                    rows_v.at[b].at[pl.ds(j * ROW, ROW)],
                    gsem[b],
                ).wait()

        for b in range(NBUF):
            fire_gathers(b, b)

        def body(g, carry):
            for b in range(NBUF):
                c = g * NBUF + b
                f0 = (row_base + c * CH) * ROW
                wait_gathers(b)
                st = pltpu.async_copy(
                    rows_v.at[b],
                    out_hbm.at[pl.ds(f0, CH * ROW)],
                    ssem[b],
                )
                st.wait()
                nxt = c + NBUF

                @pl.when(nxt < NCHUNK)
                def _():
                    fire_gathers(nxt, b)

            return carry

        lax.fori_loop(0, NITER, body, 0)

    return emb(table_p, idx)


def kernel(x, table):
    table_p = jnp.pad(table, ((0, 0), (0, PDIM - DIM)))
    idx = x.reshape(NROWS, ROW)
    out = _embed(table_p, idx)
    return out[:, :DIM].reshape(B, L, DIM)
